# baseline (speedup 1.0000x reference)
import functools
import math

import jax
import jax.numpy as jnp
from jax import lax
from jax.experimental import pallas as pl
from jax.experimental.pallas import tpu as pltpu
from jax.experimental.pallas import tpu_sc as plsc

BOOTSTRAP_FRAC = 0.4
NCORES = 2
NSUB = 16
LANES = 16


def _loss_body(p_ref, t_ref, o_ref, *, NCB):
    cb = pl.program_id(1)
    part = jnp.sum(jnp.log2(p_ref[0]) * t_ref[0], axis=0)

    @pl.when(cb == 0)
    def _init():
        o_ref[0] = part

    @pl.when((cb > 0) & (cb < NCB - 1))
    def _accum():
        o_ref[0] += part

    @pl.when(cb == NCB - 1)
    def _fin():
        o_ref[0] = 0.0 - (o_ref[0] + part)


def _find_bin(cnt_ref, sum_ref, n, ktarget):
    iota = lax.iota(jnp.int32, LANES)

    def tots(j, carry, n=n):
        tc, ts = carry
        return (tc + jnp.sum(cnt_ref[n, pl.ds(j * LANES, LANES)]),
                ts + jnp.sum(sum_ref[n, pl.ds(j * LANES, LANES)]))

    tot_c, tot_s = lax.fori_loop(
        0, 16, tots, (jnp.float32(0.0), jnp.float32(0.0)))

    def scan(j, carry, n=n):
        b, pic_b, pis_b, run_c, run_s = carry
        ch_c = cnt_ref[n, pl.ds(j * LANES, LANES)]
        ch_s = sum_ref[n, pl.ds(j * LANES, LANES)]
        pc = plsc.cumsum(ch_c) + run_c
        ps = plsc.cumsum(ch_s) + run_s
        pexcl = pc - ch_c
        m = (tot_c - pexcl) >= ktarget
        hasm = jnp.sum(m.astype(jnp.int32)) > 0
        idx = jnp.max(jnp.where(m, iota, jnp.int32(-1)))
        b = lax.select(hasm, j * LANES + idx, b)
        pic_j = jnp.sum(jnp.where(iota == idx, pc, jnp.float32(0.0)))
        pis_j = jnp.sum(jnp.where(iota == idx, ps, jnp.float32(0.0)))
        pic_b = lax.select(hasm, pic_j, pic_b)
        pis_b = lax.select(hasm, pis_j, pis_b)
        return (b, pic_b, pis_b, run_c + jnp.sum(ch_c), run_s + jnp.sum(ch_s))

    b, pic_b, pis_b, _, _ = lax.fori_loop(
        0, 16, scan,
        (jnp.int32(0), tot_c, tot_s, jnp.float32(0.0), jnp.float32(0.0)))
    return b, tot_c - pic_b, tot_s - pis_b


def _sc_select_body(loss, out, slab, hc, hs, tmp_c, tmp_s, res,
                    sh_c, sh_s, *, RPC, ROWS, k):
    s = lax.axis_index("s")
    c = lax.axis_index("c")
    row0 = c * RPC
    ones_f = jnp.ones((LANES,), jnp.float32)
    zeros_f = jnp.zeros((LANES,), jnp.float32)
    niter = slab.shape[2] // LANES

    def zero_hists():
        def zrow(r, _):
            def zch(j, _, r=r):
                sl = pl.ds(j * LANES, LANES)
                hc[r, sl] = zeros_f
                hs[r, sl] = zeros_f
                return 0
            return lax.fori_loop(0, 16, zch, 0)
        lax.fori_loop(0, RPC, zrow, 0)

    def merge_hists():
        pltpu.sync_copy(hc, sh_c.at[s])
        pltpu.sync_copy(hs, sh_s.at[s])
        plsc.subcore_barrier()
        zero_hists()
        for g in range(NSUB // 4):
            pltpu.sync_copy(sh_c.at[pl.ds(g * 4, 4)], tmp_c)
            pltpu.sync_copy(sh_s.at[pl.ds(g * 4, 4)], tmp_s)

            def mrow(n, _):
                def mch(j, _, n=n):
                    sl = pl.ds(j * LANES, LANES)

                    def acc(t, ab, n=n, sl=sl):
                        return (ab[0] + tmp_c[t, n, sl],
                                ab[1] + tmp_s[t, n, sl])

                    tc, ts = lax.fori_loop(0, 4, acc, (zeros_f, zeros_f))
                    hc[n, sl] += tc
                    hs[n, sl] += ts
                    return 0
                return lax.fori_loop(0, 16, mch, 0)
            lax.fori_loop(0, RPC, mrow, 0)
        plsc.subcore_barrier()

    for n in range(RPC):
        pltpu.sync_copy(loss.at[row0 + n, pl.ds(s * ROWS, ROWS), :],
                        slab.at[n])

    zero_hists()
    for n in range(RPC):
        def row1(r, _, n=n):
            def col1(i, _, r=r, n=n):
                v = slab[n, r, pl.ds(i * LANES, LANES)]
                rn = jnp.full((LANES,), n, jnp.int32)
                b1 = lax.bitcast_convert_type(v, jnp.int32) >> 24
                plsc.addupdate_scatter(hc, [rn, b1], ones_f)
                plsc.addupdate_scatter(hs, [rn, b1], v)
                return 0
            return lax.fori_loop(0, niter, col1, 0)
        lax.fori_loop(0, ROWS, row1, 0)
    merge_hists()

    b1s = []
    krems = []
    sabove = []
    for n in range(RPC):
        b1, cab, sab = _find_bin(hc, hs, n, jnp.float32(k))
        b1s.append(b1)
        krems.append(jnp.float32(k) - cab)
        sabove.append(sab)

    zero_hists()
    for n in range(RPC):
        def row2(r, _, n=n):
            def col2(i, _, r=r, n=n):
                v = slab[n, r, pl.ds(i * LANES, LANES)]
                rn = jnp.full((LANES,), n, jnp.int32)
                bits = lax.bitcast_convert_type(v, jnp.int32)
                match = (bits >> 24) == b1s[n]
                b2 = (bits >> 16) & 255
                plsc.addupdate_scatter(hc, [rn, b2], ones_f, mask=match)
                plsc.addupdate_scatter(hs, [rn, b2], v, mask=match)
                return 0
            return lax.fori_loop(0, niter, col2, 0)
        lax.fori_loop(0, ROWS, row2, 0)
    merge_hists()

    for n in range(RPC):
        b2, cab2, sab2 = _find_bin(hc, hs, n, krems[n])
        krem2 = krems[n] - cab2
        vmid_bits = (b1s[n] << 24) | (b2 << 16) | 0x8000
        vmid = lax.bitcast_convert_type(
            jnp.full((LANES,), vmid_bits, jnp.int32), jnp.float32)
        topk = (jnp.full((LANES,), sabove[n] + sab2)
                + jnp.full((LANES,), krem2) * vmid)
        res[pl.ds(n * LANES, LANES)] = topk

    @pl.when(s == 0)
    def _write():
        pltpu.sync_copy(res, out.at[pl.ds(row0 * LANES, RPC * LANES)])


def kernel(predict, target):
    N, C, H, W = target.shape
    k = int(H * W * BOOTSTRAP_FRAC)
    cblk = 8 if C % 8 == 0 else 1
    ncb = C // cblk
    RPC = N // NCORES
    ROWS = H // NSUB

    loss = pl.pallas_call(
        functools.partial(_loss_body, NCB=ncb),
        grid=(N, ncb),
        in_specs=[
            pl.BlockSpec((1, cblk, H, W), lambda n, c: (n, c, 0, 0)),
            pl.BlockSpec((1, cblk, H, W), lambda n, c: (n, c, 0, 0)),
        ],
        out_specs=pl.BlockSpec((1, H, W), lambda n, c: (n, 0, 0)),
        out_shape=jax.ShapeDtypeStruct((N, H, W), jnp.float32),
    )(predict, target)

    @functools.partial(
        pl.kernel,
        out_type=jax.ShapeDtypeStruct((N * LANES,), jnp.float32),
        mesh=plsc.VectorSubcoreMesh(core_axis_name="c", subcore_axis_name="s"),
        compiler_params=pltpu.CompilerParams(needs_layout_passes=False),
        scratch_types=[
            pltpu.VMEM((RPC, ROWS, W), jnp.float32),
            pltpu.VMEM((RPC, 256), jnp.float32),
            pltpu.VMEM((RPC, 256), jnp.float32),
            pltpu.VMEM((4, RPC, 256), jnp.float32),
            pltpu.VMEM((4, RPC, 256), jnp.float32),
            pltpu.VMEM((RPC * LANES,), jnp.float32),
            pltpu.VMEM_SHARED((NSUB, RPC, 256), jnp.float32),
            pltpu.VMEM_SHARED((NSUB, RPC, 256), jnp.float32),
        ],
    )
    def sel_call(loss_hbm, out_hbm, slab, hc, hs, tmp_c, tmp_s, res,
                 sh_c, sh_s):
        _sc_select_body(loss_hbm, out_hbm, slab, hc, hs, tmp_c, tmp_s, res,
                        sh_c, sh_s, RPC=RPC, ROWS=ROWS, k=k)

    sel = sel_call(loss)
    sel2 = sel.reshape(N, LANES)
    return jnp.sum(sel2[:, 0]) * jnp.float32(math.log(2.0) / (N * k))


# speedup vs baseline: 2.9212x; 2.9212x over previous
import functools
import math

import jax
import jax.numpy as jnp
from jax import lax
from jax.experimental import pallas as pl
from jax.experimental.pallas import tpu as pltpu

BOOTSTRAP_FRAC = 0.4
SEARCH_ITERS = 18


def _body(p_ref, t_ref, out_ref, acc_ref, *, N, NCB, k, scale):
    n = pl.program_id(0)
    cb = pl.program_id(1)

    part = jnp.sum(jnp.log2(p_ref[0]) * t_ref[0], axis=0)

    @pl.when(cb == 0)
    def _init_acc():
        acc_ref[n] = part

    @pl.when(cb > 0)
    def _accum():
        acc_ref[n] += part

    @pl.when((n == N - 1) & (cb == NCB - 1))
    def _select():
        for r in range(N):
            acc_ref[r] = 0.0 - acc_ref[r]

        def count_ge(r, trial):
            vb = lax.bitcast_convert_type(acc_ref[r], jnp.int32)
            return jnp.sum((vb >= trial).astype(jnp.int32))

        def step(i, bits):
            out = []
            for r in range(N):
                trial = bits[r] | (1 << (30 - i))
                out.append(lax.select(count_ge(r, trial) >= k, trial, bits[r]))
            return tuple(out)

        kbits = lax.fori_loop(0, SEARCH_ITERS, step, (jnp.int32(0),) * N)

        rem = 31 - SEARCH_ITERS
        total = jnp.float32(0.0)
        for r in range(N):
            v = acc_ref[r]
            vb = lax.bitcast_convert_type(v, jnp.int32)
            hi = kbits[r] + (1 << rem)
            vk = lax.bitcast_convert_type(
                kbits[r] + (1 << (rem - 1)), jnp.float32
            )
            ge = vb >= hi
            s_ge = jnp.sum(jnp.where(ge, v, 0.0))
            c_ge = jnp.sum(ge.astype(jnp.int32))
            total += s_ge + (k - c_ge).astype(jnp.float32) * vk

        out_ref[0, 0] = total * scale


def kernel(predict, target):
    N, C, H, W = target.shape
    k = int(H * W * BOOTSTRAP_FRAC)
    cblk = 8 if C % 8 == 0 else (4 if C % 4 == 0 else 1)
    ncb = C // cblk

    out = pl.pallas_call(
        functools.partial(
            _body, N=N, NCB=ncb, k=k, scale=math.log(2.0) / (N * k)
        ),
        grid=(N, ncb),
        in_specs=[
            pl.BlockSpec((1, cblk, H, W), lambda n, c: (n, c, 0, 0)),
            pl.BlockSpec((1, cblk, H, W), lambda n, c: (n, c, 0, 0)),
        ],
        out_specs=pl.BlockSpec(memory_space=pltpu.SMEM),
        out_shape=jax.ShapeDtypeStruct((1, 1), jnp.float32),
        scratch_shapes=[pltpu.VMEM((N, H, W), jnp.float32)],
    )(predict, target)
    return out[0, 0]
